# Initial kernel scaffold; baseline (speedup 1.0000x reference)
import jax
import jax.numpy as jnp
from jax.experimental import pallas as pl


def kernel(seq, table):
    raise NotImplementedError("write your pallas kernel here")


# speedup vs baseline: 1.6866x; 1.6866x over previous
import jax
import jax.numpy as jnp
from jax import lax
from jax.experimental import pallas as pl
from jax.experimental.pallas import tpu as pltpu
from jax.experimental.pallas import tpu_sc as plsc

HIDDEN = 64
NC, NS = 2, 16
NW = NC * NS
CHUNK = 128
TOTAL = 16384 * 50
PER_W = TOTAL // NW
NCH = PER_W // CHUNK


def _gather_body(seq_hbm, table_hbm, out_hbm, idx_v, rows_v, sem):
    wid = lax.axis_index("s") * NC + lax.axis_index("c")
    pltpu.sync_copy(seq_hbm.at[wid], idx_v)

    def step(j, carry):
        pltpu.async_copy(table_hbm.at[idx_v.at[j]], rows_v, sem).wait()
        pltpu.sync_copy(rows_v, out_hbm.at[wid, j])
        return carry

    lax.fori_loop(0, NCH, step, 0)


def kernel(seq, table):
    seq3 = seq.reshape(NW, NCH, CHUNK)
    out = pl.kernel(
        _gather_body,
        out_type=jax.ShapeDtypeStruct((NW, NCH, CHUNK, HIDDEN), jnp.float32),
        mesh=plsc.VectorSubcoreMesh(core_axis_name="c", subcore_axis_name="s"),
        scratch_types=[
            pltpu.VMEM((NCH, CHUNK), jnp.int32),
            pltpu.VMEM((CHUNK, HIDDEN), jnp.float32),
            pltpu.SemaphoreType.DMA,
        ],
        compiler_params=pltpu.CompilerParams(use_tc_tiling_on_sc=False),
    )(seq3, table)
    return out.reshape(16384, 50, HIDDEN)


# speedup vs baseline: 1.8715x; 1.1096x over previous
import jax
import jax.numpy as jnp
from jax import lax
from jax.experimental import pallas as pl
from jax.experimental.pallas import tpu as pltpu
from jax.experimental.pallas import tpu_sc as plsc

HIDDEN = 64
NC, NS = 2, 16
NW = NC * NS
CHUNK = 128
TOTAL = 16384 * 50
PER_W = TOTAL // NW
NCH = PER_W // CHUNK
K = 4
NR = NCH // K


def _gather_body(seq_hbm, table_hbm, out_hbm, idx_v, rows_v, gsem, osem):
    wid = lax.axis_index("s") * NC + lax.axis_index("c")
    pltpu.sync_copy(seq_hbm.at[wid], idx_v)

    def g_copy(g, r, k):
        j = r * K + k
        return pltpu.make_async_copy(
            table_hbm.at[idx_v.at[j]], rows_v.at[g, k], gsem.at[g])

    def o_copy(g, r, k):
        j = r * K + k
        return pltpu.make_async_copy(
            rows_v.at[g, k], out_hbm.at[wid, j], osem.at[g])

    for k in range(K):
        g_copy(0, 0, k).start()

    def round_body(r, carry):
        g = r % 2
        g2 = 1 - g
        for k in range(K):
            g_copy(g, r, k).wait()

        @pl.when(r + 1 < NR)
        def _fire_next():
            @pl.when(r >= 1)
            def _drain_old():
                for k in range(K):
                    o_copy(g2, r - 1, k).wait()
            for k in range(K):
                g_copy(g2, r + 1, k).start()

        for k in range(K):
            o_copy(g, r, k).start()
        return carry

    lax.fori_loop(0, NR, round_body, 0)
    for k in range(K):
        o_copy((NR - 2) % 2, NR - 2, k).wait()
    for k in range(K):
        o_copy((NR - 1) % 2, NR - 1, k).wait()


def kernel(seq, table):
    seq3 = seq.reshape(NW, NCH, CHUNK)
    out = pl.kernel(
        _gather_body,
        out_type=jax.ShapeDtypeStruct((NW, NCH, CHUNK, HIDDEN), jnp.float32),
        mesh=plsc.VectorSubcoreMesh(core_axis_name="c", subcore_axis_name="s"),
        scratch_types=[
            pltpu.VMEM((NCH, CHUNK), jnp.int32),
            pltpu.VMEM((2, K, CHUNK, HIDDEN), jnp.float32),
            pltpu.SemaphoreType.DMA((2,)),
            pltpu.SemaphoreType.DMA((2,)),
        ],
        compiler_params=pltpu.CompilerParams(use_tc_tiling_on_sc=False),
    )(seq3, table)
    return out.reshape(16384, 50, HIDDEN)


# speedup vs baseline: 1.8728x; 1.0007x over previous
import jax
import jax.numpy as jnp
from jax import lax
from jax.experimental import pallas as pl
from jax.experimental.pallas import tpu as pltpu
from jax.experimental.pallas import tpu_sc as plsc

HIDDEN = 64
NC, NS = 2, 16
NW = NC * NS
CHUNK = 512
TOTAL = 16384 * 50
PER_W = TOTAL // NW
NCH = PER_W // CHUNK
K = 1
NR = NCH // K


def _gather_body(seq_hbm, table_hbm, out_hbm, idx_v, rows_v, gsem, osem):
    wid = lax.axis_index("s") * NC + lax.axis_index("c")
    pltpu.sync_copy(seq_hbm.at[wid], idx_v)

    def g_copy(g, r):
        return pltpu.make_async_copy(
            table_hbm.at[idx_v.at[r]], rows_v.at[g], gsem.at[g])

    def o_copy(g, r):
        return pltpu.make_async_copy(
            rows_v.at[g], out_hbm.at[wid, r], osem.at[g])

    g_copy(0, 0).start()

    def round_body(r, carry):
        g = r % 2
        g2 = 1 - g
        g_copy(g, r).wait()

        @pl.when(r + 1 < NR)
        def _fire_next():
            @pl.when(r >= 1)
            def _drain_old():
                o_copy(g2, r - 1).wait()
            g_copy(g2, r + 1).start()

        o_copy(g, r).start()
        return carry

    lax.fori_loop(0, NR, round_body, 0)
    o_copy((NR - 2) % 2, NR - 2).wait()
    o_copy((NR - 1) % 2, NR - 1).wait()


def kernel(seq, table):
    seq3 = seq.reshape(NW, NCH, CHUNK)
    out = pl.kernel(
        _gather_body,
        out_type=jax.ShapeDtypeStruct((NW, NCH, CHUNK, HIDDEN), jnp.float32),
        mesh=plsc.VectorSubcoreMesh(core_axis_name="c", subcore_axis_name="s"),
        scratch_types=[
            pltpu.VMEM((NCH, CHUNK), jnp.int32),
            pltpu.VMEM((2, CHUNK, HIDDEN), jnp.float32),
            pltpu.SemaphoreType.DMA((2,)),
            pltpu.SemaphoreType.DMA((2,)),
        ],
        compiler_params=pltpu.CompilerParams(use_tc_tiling_on_sc=False),
    )(seq3, table)
    return out.reshape(16384, 50, HIDDEN)
